# baseline (speedup 1.0000x reference)
import functools

import jax
import jax.numpy as jnp
from jax import lax
from jax.experimental import pallas as pl
from jax.experimental.pallas import tpu as pltpu

_E = 8
_K = 2
_D = 768
_F = 3072
_BLK = 128
_INTERPRET = False


def _router_body(tok_ref, wg_ref, pos_ref, gate_ref, blk_ref, *, nb):
    logits = jnp.dot(tok_ref[...], wg_ref[...], preferred_element_type=jnp.float32)
    t, c = logits.shape
    col = jax.lax.broadcasted_iota(jnp.int32, (t, c), 1)
    lg = jnp.where(col < _E, logits, -1e30)
    m1 = jnp.max(lg, axis=1, keepdims=True)
    i1 = jnp.min(jnp.where(lg == m1, col, c), axis=1, keepdims=True)
    lg2 = jnp.where(col == i1, -1e30, lg)
    m2 = jnp.max(lg2, axis=1, keepdims=True)
    i2 = jnp.min(jnp.where(lg2 == m2, col, c), axis=1, keepdims=True)
    g1 = 1.0 / (1.0 + jnp.exp(m2 - m1))
    gate_ref[...] = jnp.where(col == 0, g1, jnp.where(col == 1, 1.0 - g1, 0.0))

    oh2 = ((col == i1) | (col == i2)).astype(jnp.float32)
    counts = jnp.sum(oh2, axis=0, keepdims=True)
    nblk = jnp.floor((counts + (_BLK - 1)) * (1.0 / _BLK))
    lane_r = jax.lax.broadcasted_iota(jnp.int32, (c, c), 0)
    lane_c = jax.lax.broadcasted_iota(jnp.int32, (c, c), 1)
    upper = (lane_r < lane_c).astype(jnp.float32)
    blk_start = jnp.dot(nblk, upper, preferred_element_type=jnp.float32)
    off = blk_start * float(_BLK)

    row_r = jax.lax.broadcasted_iota(jnp.int32, (t, t), 0)
    row_c = jax.lax.broadcasted_iota(jnp.int32, (t, t), 1)
    lstrict = (row_c < row_r).astype(jnp.float32)
    rank = jnp.dot(lstrict, oh2, preferred_element_type=jnp.float32)

    sel1 = (col == i1).astype(jnp.float32)
    sel2 = (col == i2).astype(jnp.float32)
    offb = jnp.broadcast_to(off, (t, c))
    pos1 = jnp.sum(sel1 * (offb + rank), axis=1, keepdims=True)
    pos2 = jnp.sum(sel2 * (offb + rank), axis=1, keepdims=True)
    pos_ref[...] = jnp.where(col == 0, pos1, jnp.where(col == 1, pos2, 0.0)
                             ).astype(jnp.int32)

    incl = (blk_start + nblk).astype(jnp.int32)
    jrow = jax.lax.broadcasted_iota(jnp.int32, (nb, c), 0)
    inclb = jnp.broadcast_to(incl, (nb, c))
    lane2 = jax.lax.broadcasted_iota(jnp.int32, (nb, c), 1)
    hit = ((inclb <= jrow) & (lane2 < _E)).astype(jnp.int32)
    blk_ref[...] = jnp.minimum(jnp.sum(hit, axis=1, keepdims=True), _E - 1
                               ) + jnp.zeros((nb, c), jnp.int32)


def _sc_gather(table, idx):
    from jax.experimental.pallas import tpu_sc as plsc

    (b,) = idx.shape
    _, d = table.shape
    info = plsc.get_sparse_core_info()
    nw = info.num_cores * info.num_subcores
    bw = b // nw
    mesh = plsc.VectorSubcoreMesh(core_axis_name="c", subcore_axis_name="s")

    @functools.partial(
        pl.kernel, mesh=mesh,
        out_type=jax.ShapeDtypeStruct((b, d), jnp.float32),
        scratch_types=[
            pltpu.VMEM((bw,), jnp.int32),
            pltpu.VMEM((bw, d), jnp.float32),
            pltpu.SemaphoreType.DMA,
        ],
    )
    def k(table_hbm, idx_hbm, out_hbm, idx_v, rows_v, sem):
        wid = lax.axis_index("s") * info.num_cores + lax.axis_index("c")
        base = wid * bw
        pltpu.sync_copy(idx_hbm.at[pl.ds(base, bw)], idx_v)
        pltpu.async_copy(table_hbm.at[idx_v], rows_v, sem).wait()
        pltpu.sync_copy(rows_v, out_hbm.at[pl.ds(base, bw)])

    return k(table, idx)


def _ffn_body(blke_ref, xs_ref, w1_ref, b1_ref, w2_ref, b2_ref, ys_ref):
    h = jnp.dot(xs_ref[...], w1_ref[0], preferred_element_type=jnp.float32)
    h = jnp.maximum(h + b1_ref[0], 0.0)
    ys_ref[...] = jnp.dot(h, w2_ref[0], preferred_element_type=jnp.float32
                          ) + b2_ref[0]


def _combine_body(x_ref, yp_ref, gate_ref, lng_ref, lnb_ref, out_ref, *, t, tp):
    g = gate_ref[...]
    y0 = yp_ref[pl.ds(0, t), :]
    y1 = yp_ref[pl.ds(tp, t), :]
    a = x_ref[...] + g[:, 0:1] * y0 + g[:, 1:2] * y1
    m = jnp.mean(a, axis=1, keepdims=True)
    v = jnp.mean((a - m) * (a - m), axis=1, keepdims=True)
    out_ref[...] = (a - m) * jax.lax.rsqrt(v + 1e-5) * lng_ref[...] + lnb_ref[...]


def kernel(x, Wg, W1, b1, W2, b2, ln_g, ln_b):
    bv, nv, dv = x.shape
    t = bv * nv
    p = t * _K
    nb = (p + _BLK - 1) // _BLK + (_E - 1)
    nr = nb * _BLK
    tp = ((t + _BLK - 1) // _BLK) * _BLK

    tokens = x.reshape(t, dv)

    wgp = jnp.zeros((dv, 128), Wg.dtype).at[:, :_E].set(Wg)
    pos_out, gate_out, blk_out = pl.pallas_call(
        functools.partial(_router_body, nb=nb),
        out_shape=(
            jax.ShapeDtypeStruct((t, 128), jnp.int32),
            jax.ShapeDtypeStruct((t, 128), jnp.float32),
            jax.ShapeDtypeStruct((nb, 128), jnp.int32),
        ),
        interpret=_INTERPRET,
    )(tokens, wgp)

    pos0 = pos_out[:, 0]
    pos1 = pos_out[:, 1]
    blk_e = blk_out[:, 0]
    zpad = jnp.zeros((tp - t,), jnp.int32)
    posq = jnp.concatenate([pos0, zpad, pos1, zpad])
    pos_flat = jnp.concatenate([pos0, pos1])
    tokid = jnp.arange(p, dtype=jnp.int32) % t
    row_token = jnp.full((nr,), t - 1, jnp.int32).at[pos_flat].set(tokid)

    xs = _sc_gather(tokens, row_token)

    b1r = b1.reshape(_E, 1, _F)
    b2r = b2.reshape(_E, 1, dv)
    grid_spec = pltpu.PrefetchScalarGridSpec(
        num_scalar_prefetch=1,
        grid=(nb,),
        in_specs=[
            pl.BlockSpec((_BLK, dv), lambda i, be: (i, 0)),
            pl.BlockSpec((1, dv, _F), lambda i, be: (be[i], 0, 0)),
            pl.BlockSpec((1, 1, _F), lambda i, be: (be[i], 0, 0)),
            pl.BlockSpec((1, _F, dv), lambda i, be: (be[i], 0, 0)),
            pl.BlockSpec((1, 1, dv), lambda i, be: (be[i], 0, 0)),
        ],
        out_specs=pl.BlockSpec((_BLK, dv), lambda i, be: (i, 0)),
    )
    ys = pl.pallas_call(
        _ffn_body,
        grid_spec=grid_spec,
        out_shape=jax.ShapeDtypeStruct((nr, dv), jnp.float32),
        compiler_params=pltpu.CompilerParams(
            dimension_semantics=("arbitrary",)),
        interpret=_INTERPRET,
    )(blk_e, xs, W1, b1r, W2, b2r)

    yp = _sc_gather(ys, posq)

    out = pl.pallas_call(
        functools.partial(_combine_body, t=t, tp=tp),
        out_shape=jax.ShapeDtypeStruct((t, dv), jnp.float32),
        interpret=_INTERPRET,
    )(tokens, yp, gate_out, ln_g.reshape(1, dv), ln_b.reshape(1, dv))

    return out.reshape(bv, nv, dv)


# speedup vs baseline: 1.2080x; 1.2080x over previous
import functools

import jax
import jax.numpy as jnp
from jax.experimental import pallas as pl
from jax.experimental.pallas import tpu as pltpu

_E = 8
_K = 2
_D = 768
_F = 3072
_BLK = 128
_INTERPRET = False


def _router_body(tok_ref, wg_ref, pos_ref, gate_ref, blk_ref, *, nb):
    logits = jnp.dot(tok_ref[...], wg_ref[...], preferred_element_type=jnp.float32)
    t, c = logits.shape
    col = jax.lax.broadcasted_iota(jnp.int32, (t, c), 1)
    lg = jnp.where(col < _E, logits, -1e30)
    m1 = jnp.max(lg, axis=1, keepdims=True)
    i1 = jnp.min(jnp.where(lg == m1, col, c), axis=1, keepdims=True)
    lg2 = jnp.where(col == i1, -1e30, lg)
    m2 = jnp.max(lg2, axis=1, keepdims=True)
    i2 = jnp.min(jnp.where(lg2 == m2, col, c), axis=1, keepdims=True)
    g1 = 1.0 / (1.0 + jnp.exp(m2 - m1))
    gate_ref[...] = jnp.where(col == 0, g1, jnp.where(col == 1, 1.0 - g1, 0.0))

    oh2 = ((col == i1) | (col == i2)).astype(jnp.float32)
    counts = jnp.sum(oh2, axis=0, keepdims=True)
    nblk = jnp.floor((counts + (_BLK - 1)) * (1.0 / _BLK))
    lane_r = jax.lax.broadcasted_iota(jnp.int32, (c, c), 0)
    lane_c = jax.lax.broadcasted_iota(jnp.int32, (c, c), 1)
    upper = (lane_r < lane_c).astype(jnp.float32)
    blk_start = jnp.dot(nblk, upper, preferred_element_type=jnp.float32)
    off = blk_start * float(_BLK)

    row_r = jax.lax.broadcasted_iota(jnp.int32, (t, t), 0)
    row_c = jax.lax.broadcasted_iota(jnp.int32, (t, t), 1)
    lstrict = (row_c < row_r).astype(jnp.float32)
    rank = jnp.dot(lstrict, oh2, preferred_element_type=jnp.float32)

    sel1 = (col == i1).astype(jnp.float32)
    sel2 = (col == i2).astype(jnp.float32)
    offb = jnp.broadcast_to(off, (t, c))
    pos1 = jnp.sum(sel1 * (offb + rank), axis=1, keepdims=True)
    pos2 = jnp.sum(sel2 * (offb + rank), axis=1, keepdims=True)
    pos_ref[...] = jnp.where(col == 0, pos1, jnp.where(col == 1, pos2, 0.0)
                             ).astype(jnp.int32)

    incl = (blk_start + nblk).astype(jnp.int32)
    jrow = jax.lax.broadcasted_iota(jnp.int32, (nb, c), 0)
    inclb = jnp.broadcast_to(incl, (nb, c))
    lane2 = jax.lax.broadcasted_iota(jnp.int32, (nb, c), 1)
    hit = ((inclb <= jrow) & (lane2 < _E)).astype(jnp.int32)
    blk_ref[...] = jnp.minimum(jnp.sum(hit, axis=1, keepdims=True), _E - 1
                               ) + jnp.zeros((nb, c), jnp.int32)


def _ffn_body(rowtok_ref, rowgate_ref, blke_ref,
              tok_ref, w1_ref, b1_ref, w2_ref, b2_ref, lng_ref, lnb_ref,
              acc_ref, xb_ref, yb_ref, w1b_ref, w2b_ref, *, nb, t):
    i = pl.program_id(0)

    @pl.when(i == 0)
    def _():
        acc_ref[...] = tok_ref[...]

    prev = blke_ref[jnp.maximum(i - 1, 0)]

    @pl.when((i == 0) | (blke_ref[i] != prev))
    def _():
        w1b_ref[...] = w1_ref[0].astype(jnp.bfloat16)
        w2b_ref[...] = w2_ref[0].astype(jnp.bfloat16)

    def gather(r, _):
        tk = jnp.minimum(rowtok_ref[i * _BLK + r], t - 1)
        xb_ref[r, :] = tok_ref[tk, :]
        return 0

    jax.lax.fori_loop(0, _BLK, gather, 0, unroll=8)

    h = jnp.dot(xb_ref[...].astype(jnp.bfloat16), w1b_ref[...],
                preferred_element_type=jnp.float32)
    h = jnp.maximum(h + b1_ref[0], 0.0)
    y = jnp.dot(h.astype(jnp.bfloat16), w2b_ref[...],
                preferred_element_type=jnp.float32) + b2_ref[0]
    yb_ref[...] = y

    def scatter(r, _):
        tk = jnp.minimum(rowtok_ref[i * _BLK + r], t - 1)
        g = rowgate_ref[i * _BLK + r]
        acc_ref[tk, :] = acc_ref[tk, :] + g * yb_ref[r, :]
        return 0

    jax.lax.fori_loop(0, _BLK, scatter, 0, unroll=8)

    @pl.when(i == nb - 1)
    def _():
        a = acc_ref[...]
        m = jnp.mean(a, axis=1, keepdims=True)
        v = jnp.mean((a - m) * (a - m), axis=1, keepdims=True)
        acc_ref[...] = (a - m) * jax.lax.rsqrt(v + 1e-5) * lng_ref[...] + lnb_ref[...]


def kernel(x, Wg, W1, b1, W2, b2, ln_g, ln_b):
    bv, nv, dv = x.shape
    t = bv * nv
    p = t * _K
    nb = (p + _BLK - 1) // _BLK + (_E - 1)
    nr = nb * _BLK

    tokens = x.reshape(t, dv)

    wgp = jnp.zeros((dv, 128), Wg.dtype).at[:, :_E].set(Wg)
    pos_out, gate_out, blk_out = pl.pallas_call(
        functools.partial(_router_body, nb=nb),
        out_shape=(
            jax.ShapeDtypeStruct((t, 128), jnp.int32),
            jax.ShapeDtypeStruct((t, 128), jnp.float32),
            jax.ShapeDtypeStruct((nb, 128), jnp.int32),
        ),
        interpret=_INTERPRET,
    )(tokens, wgp)

    pos2 = pos_out[:, :_K].reshape(p)
    gf = gate_out[:, :_K].reshape(p)
    blk_e = blk_out[:, 0]
    row_token = jnp.full((nr,), t, jnp.int32).at[pos2].set(
        jnp.arange(p, dtype=jnp.int32) // _K)
    row_gate = jnp.zeros((nr,), jnp.float32).at[pos2].set(gf)

    b1r = b1.reshape(_E, 1, _F)
    b2r = b2.reshape(_E, 1, dv)
    lngr = ln_g.reshape(1, dv)
    lnbr = ln_b.reshape(1, dv)

    grid_spec = pltpu.PrefetchScalarGridSpec(
        num_scalar_prefetch=3,
        grid=(nb,),
        in_specs=[
            pl.BlockSpec((t, dv), lambda i, rt, rg, be: (0, 0)),
            pl.BlockSpec((1, dv, _F), lambda i, rt, rg, be: (be[i], 0, 0)),
            pl.BlockSpec((1, 1, _F), lambda i, rt, rg, be: (be[i], 0, 0)),
            pl.BlockSpec((1, _F, dv), lambda i, rt, rg, be: (be[i], 0, 0)),
            pl.BlockSpec((1, 1, dv), lambda i, rt, rg, be: (be[i], 0, 0)),
            pl.BlockSpec((1, dv), lambda i, rt, rg, be: (0, 0)),
            pl.BlockSpec((1, dv), lambda i, rt, rg, be: (0, 0)),
        ],
        out_specs=pl.BlockSpec((t, dv), lambda i, rt, rg, be: (0, 0)),
        scratch_shapes=[
            pltpu.VMEM((_BLK, dv), jnp.float32),
            pltpu.VMEM((_BLK, dv), jnp.float32),
            pltpu.VMEM((dv, _F), jnp.bfloat16),
            pltpu.VMEM((_F, dv), jnp.bfloat16),
        ],
    )
    acc = pl.pallas_call(
        functools.partial(_ffn_body, nb=nb, t=t),
        grid_spec=grid_spec,
        out_shape=jax.ShapeDtypeStruct((t, dv), jnp.float32),
        compiler_params=pltpu.CompilerParams(
            dimension_semantics=("arbitrary",)),
        interpret=_INTERPRET,
    )(row_token, row_gate, blk_e,
      tokens, W1, b1r, W2, b2r, lngr, lnbr)

    return acc.reshape(bv, nv, dv)


# speedup vs baseline: 1.2146x; 1.0054x over previous
import functools

import jax
import jax.numpy as jnp
from jax.experimental import pallas as pl
from jax.experimental.pallas import tpu as pltpu

_E = 8
_K = 2
_D = 768
_F = 3072
_BLK = 256
_INTERPRET = False


def _router_body(tok_ref, wg_ref, pos_ref, gate_ref, blk_ref, *, nb):
    logits = jnp.dot(tok_ref[...], wg_ref[...], preferred_element_type=jnp.float32)
    t, c = logits.shape
    col = jax.lax.broadcasted_iota(jnp.int32, (t, c), 1)
    lg = jnp.where(col < _E, logits, -1e30)
    m1 = jnp.max(lg, axis=1, keepdims=True)
    i1 = jnp.min(jnp.where(lg == m1, col, c), axis=1, keepdims=True)
    lg2 = jnp.where(col == i1, -1e30, lg)
    m2 = jnp.max(lg2, axis=1, keepdims=True)
    i2 = jnp.min(jnp.where(lg2 == m2, col, c), axis=1, keepdims=True)
    g1 = 1.0 / (1.0 + jnp.exp(m2 - m1))
    gate_ref[...] = jnp.where(col == 0, g1, jnp.where(col == 1, 1.0 - g1, 0.0))

    oh2 = ((col == i1) | (col == i2)).astype(jnp.float32)
    counts = jnp.sum(oh2, axis=0, keepdims=True)
    nblk = jnp.floor((counts + (_BLK - 1)) * (1.0 / _BLK))
    lane_r = jax.lax.broadcasted_iota(jnp.int32, (c, c), 0)
    lane_c = jax.lax.broadcasted_iota(jnp.int32, (c, c), 1)
    upper = (lane_r < lane_c).astype(jnp.float32)
    blk_start = jnp.dot(nblk, upper, preferred_element_type=jnp.float32)
    off = blk_start * float(_BLK)

    row_r = jax.lax.broadcasted_iota(jnp.int32, (t, t), 0)
    row_c = jax.lax.broadcasted_iota(jnp.int32, (t, t), 1)
    lstrict = (row_c < row_r).astype(jnp.float32)
    rank = jnp.dot(lstrict, oh2, preferred_element_type=jnp.float32)

    sel1 = (col == i1).astype(jnp.float32)
    sel2 = (col == i2).astype(jnp.float32)
    offb = jnp.broadcast_to(off, (t, c))
    pos1 = jnp.sum(sel1 * (offb + rank), axis=1, keepdims=True)
    pos2 = jnp.sum(sel2 * (offb + rank), axis=1, keepdims=True)
    pos_ref[...] = jnp.where(col == 0, pos1, jnp.where(col == 1, pos2, 0.0)
                             ).astype(jnp.int32)

    incl = (blk_start + nblk).astype(jnp.int32)
    jrow = jax.lax.broadcasted_iota(jnp.int32, (nb, c), 0)
    inclb = jnp.broadcast_to(incl, (nb, c))
    lane2 = jax.lax.broadcasted_iota(jnp.int32, (nb, c), 1)
    hit = ((inclb <= jrow) & (lane2 < _E)).astype(jnp.int32)
    blk_ref[...] = jnp.minimum(jnp.sum(hit, axis=1, keepdims=True), _E - 1
                               ) + jnp.zeros((nb, c), jnp.int32)


def _ffn_body(rowtok_ref, rowgate_ref, blke_ref,
              tok_ref, w1a_ref, w1b_ref, b1a_ref, b1b_ref,
              w2a_ref, w2b_ref, b2_ref,
              lng_ref, lnb_ref, acc_ref, xb_ref, yb_ref, *, nb, t):
    i = pl.program_id(0)

    @pl.when(i == 0)
    def _():
        acc_ref[...] = tok_ref[...]

    def gather(r, _):
        tk = jnp.minimum(rowtok_ref[i * _BLK + r], t - 1)
        xb_ref[r, :] = tok_ref[tk, :]
        return 0

    jax.lax.fori_loop(0, _BLK, gather, 0, unroll=16)

    xb = xb_ref[...]
    h1 = jnp.maximum(
        jnp.dot(xb, w1a_ref[0], preferred_element_type=jnp.float32)
        + b1a_ref[0], 0.0)
    h2 = jnp.maximum(
        jnp.dot(xb, w1b_ref[0], preferred_element_type=jnp.float32)
        + b1b_ref[0], 0.0)
    y = (jnp.dot(h1, w2a_ref[0], preferred_element_type=jnp.float32)
         + jnp.dot(h2, w2b_ref[0], preferred_element_type=jnp.float32)
         + b2_ref[0])
    yb_ref[...] = y

    def scatter(r, _):
        tk = jnp.minimum(rowtok_ref[i * _BLK + r], t - 1)
        g = rowgate_ref[i * _BLK + r]
        acc_ref[tk, :] = acc_ref[tk, :] + g * yb_ref[r, :]
        return 0

    jax.lax.fori_loop(0, _BLK, scatter, 0, unroll=16)

    @pl.when(i == nb - 1)
    def _():
        a = acc_ref[...]
        m = jnp.mean(a, axis=1, keepdims=True)
        v = jnp.mean((a - m) * (a - m), axis=1, keepdims=True)
        acc_ref[...] = (a - m) * jax.lax.rsqrt(v + 1e-5) * lng_ref[...] + lnb_ref[...]


def kernel(x, Wg, W1, b1, W2, b2, ln_g, ln_b):
    bv, nv, dv = x.shape
    t = bv * nv
    p = t * _K
    nb = (p + _BLK - 1) // _BLK + (_E - 1)
    nr = nb * _BLK

    tokens = x.reshape(t, dv)

    wgp = jnp.zeros((dv, 128), Wg.dtype).at[:, :_E].set(Wg)
    pos_out, gate_out, blk_out = pl.pallas_call(
        functools.partial(_router_body, nb=nb),
        out_shape=(
            jax.ShapeDtypeStruct((t, 128), jnp.int32),
            jax.ShapeDtypeStruct((t, 128), jnp.float32),
            jax.ShapeDtypeStruct((nb, 128), jnp.int32),
        ),
        interpret=_INTERPRET,
    )(tokens, wgp)

    pos2 = pos_out[:, :_K].reshape(p)
    gf = gate_out[:, :_K].reshape(p)
    blk_e = blk_out[:, 0]
    row_token = jnp.full((nr,), t, jnp.int32).at[pos2].set(
        jnp.arange(p, dtype=jnp.int32) // _K)
    row_gate = jnp.zeros((nr,), jnp.float32).at[pos2].set(gf)

    fh = _F // 2
    b1r = b1.reshape(_E, 1, _F)
    b2r = b2.reshape(_E, 1, dv)
    lngr = ln_g.reshape(1, dv)
    lnbr = ln_b.reshape(1, dv)

    grid_spec = pltpu.PrefetchScalarGridSpec(
        num_scalar_prefetch=3,
        grid=(nb,),
        in_specs=[
            pl.BlockSpec((t, dv), lambda i, rt, rg, be: (0, 0)),
            pl.BlockSpec((1, dv, fh), lambda i, rt, rg, be: (be[i], 0, 0)),
            pl.BlockSpec((1, dv, fh), lambda i, rt, rg, be: (be[i], 0, 1)),
            pl.BlockSpec((1, 1, fh), lambda i, rt, rg, be: (be[i], 0, 0)),
            pl.BlockSpec((1, 1, fh), lambda i, rt, rg, be: (be[i], 0, 1)),
            pl.BlockSpec((1, fh, dv), lambda i, rt, rg, be: (be[i], 0, 0)),
            pl.BlockSpec((1, fh, dv), lambda i, rt, rg, be: (be[i], 1, 0)),
            pl.BlockSpec((1, 1, dv), lambda i, rt, rg, be: (be[i], 0, 0)),
            pl.BlockSpec((1, dv), lambda i, rt, rg, be: (0, 0)),
            pl.BlockSpec((1, dv), lambda i, rt, rg, be: (0, 0)),
        ],
        out_specs=pl.BlockSpec((t, dv), lambda i, rt, rg, be: (0, 0)),
        scratch_shapes=[
            pltpu.VMEM((_BLK, dv), jnp.float32),
            pltpu.VMEM((_BLK, dv), jnp.float32),
        ],
    )
    acc = pl.pallas_call(
        functools.partial(_ffn_body, nb=nb, t=t),
        grid_spec=grid_spec,
        out_shape=jax.ShapeDtypeStruct((t, dv), jnp.float32),
        compiler_params=pltpu.CompilerParams(
            dimension_semantics=("arbitrary",)),
        interpret=_INTERPRET,
    )(row_token, row_gate, blk_e,
      tokens, W1, W1, b1r, b1r, W2, W2, b2r, lngr, lnbr)

    return acc.reshape(bv, nv, dv)


# speedup vs baseline: 1.3294x; 1.0945x over previous
import functools

import jax
import jax.numpy as jnp
from jax.experimental import pallas as pl
from jax.experimental.pallas import tpu as pltpu

_E = 8
_K = 2
_D = 768
_F = 3072
_BLK = 128
_GQ = 65535.0
_INTERPRET = False


def _router_body(tok_ref, wg_ref, pos_ref, packed_ref, blk_ref, *, nb):
    logits = jnp.dot(tok_ref[...], wg_ref[...], preferred_element_type=jnp.float32)
    t, c = logits.shape
    col = jax.lax.broadcasted_iota(jnp.int32, (t, c), 1)
    lg = jnp.where(col < _E, logits, -1e30)
    m1 = jnp.max(lg, axis=1, keepdims=True)
    i1 = jnp.min(jnp.where(lg == m1, col, c), axis=1, keepdims=True)
    lg2 = jnp.where(col == i1, -1e30, lg)
    m2 = jnp.max(lg2, axis=1, keepdims=True)
    i2 = jnp.min(jnp.where(lg2 == m2, col, c), axis=1, keepdims=True)
    g1 = 1.0 / (1.0 + jnp.exp(m2 - m1))

    row = jax.lax.broadcasted_iota(jnp.int32, (t, c), 0)
    q1 = jnp.round(g1 * _GQ).astype(jnp.int32)
    q2 = jnp.round((1.0 - g1) * _GQ).astype(jnp.int32)
    packed_ref[...] = row * 65536 + jnp.where(col == 0, q1,
                                              jnp.where(col == 1, q2, 0))

    oh2 = ((col == i1) | (col == i2)).astype(jnp.float32)
    counts = jnp.sum(oh2, axis=0, keepdims=True)
    nblk = jnp.floor((counts + (_BLK - 1)) * (1.0 / _BLK))
    lane_r = jax.lax.broadcasted_iota(jnp.int32, (c, c), 0)
    lane_c = jax.lax.broadcasted_iota(jnp.int32, (c, c), 1)
    upper = (lane_r < lane_c).astype(jnp.float32)
    blk_start = jnp.dot(nblk, upper, preferred_element_type=jnp.float32)
    off = blk_start * float(_BLK)

    inc = oh2
    k = 1
    while k < t:
        shifted = jnp.concatenate(
            [jnp.zeros((k, c), jnp.float32), inc[: t - k]], axis=0)
        inc = inc + shifted
        k *= 2
    rank = inc - oh2

    sel1 = (col == i1).astype(jnp.float32)
    sel2 = (col == i2).astype(jnp.float32)
    offb = jnp.broadcast_to(off, (t, c))
    pos1 = jnp.sum(sel1 * (offb + rank), axis=1, keepdims=True)
    pos2 = jnp.sum(sel2 * (offb + rank), axis=1, keepdims=True)
    pos_ref[...] = jnp.where(col == 0, pos1, jnp.where(col == 1, pos2, 0.0)
                             ).astype(jnp.int32)

    incl = (blk_start + nblk).astype(jnp.int32)
    jrow = jax.lax.broadcasted_iota(jnp.int32, (nb, c), 0)
    inclb = jnp.broadcast_to(incl, (nb, c))
    lane2 = jax.lax.broadcasted_iota(jnp.int32, (nb, c), 1)
    hit = ((inclb <= jrow) & (lane2 < _E)).astype(jnp.int32)
    blk_ref[...] = jnp.minimum(jnp.sum(hit, axis=1, keepdims=True), _E - 1
                               ) + jnp.zeros((nb, c), jnp.int32)


def _ffn_body(tbl_ref, blke_ref,
              tok_ref, w1_ref, b1_ref, w2_ref, b2_ref, lng_ref, lnb_ref,
              acc_ref, xb_ref, yb_ref, *, nb, t):
    i = pl.program_id(0)

    @pl.when(i == 0)
    def _():
        acc_ref[...] = tok_ref[...]

    def gather(r, _):
        tk = jax.lax.shift_right_logical(tbl_ref[i * _BLK + r], 16)
        xb_ref[r, :] = tok_ref[tk, :]
        return 0

    jax.lax.fori_loop(0, _BLK, gather, 0, unroll=8)

    h = jnp.dot(xb_ref[...], w1_ref[0], preferred_element_type=jnp.float32)
    h = jnp.maximum(h + b1_ref[0], 0.0)
    y = jnp.dot(h, w2_ref[0], preferred_element_type=jnp.float32) + b2_ref[0]
    yb_ref[...] = y

    def scatter(r, _):
        v = tbl_ref[i * _BLK + r]
        tk = jax.lax.shift_right_logical(v, 16)
        g = (v & 65535).astype(jnp.float32) * (1.0 / _GQ)
        acc_ref[tk, :] = acc_ref[tk, :] + g * yb_ref[r, :]
        return 0

    jax.lax.fori_loop(0, _BLK, scatter, 0, unroll=8)

    @pl.when(i == nb - 1)
    def _():
        a = acc_ref[...]
        m = jnp.mean(a, axis=1, keepdims=True)
        v = jnp.mean((a - m) * (a - m), axis=1, keepdims=True)
        acc_ref[...] = (a - m) * jax.lax.rsqrt(v + 1e-5) * lng_ref[...] + lnb_ref[...]


def kernel(x, Wg, W1, b1, W2, b2, ln_g, ln_b):
    bv, nv, dv = x.shape
    t = bv * nv
    p = t * _K
    nb = (p + _BLK - 1) // _BLK + (_E - 1)
    nr = nb * _BLK

    tokens = x.reshape(t, dv)

    wgp = jnp.zeros((dv, 128), Wg.dtype).at[:, :_E].set(Wg)
    pos_out, packed_out, blk_out = pl.pallas_call(
        functools.partial(_router_body, nb=nb),
        out_shape=(
            jax.ShapeDtypeStruct((t, 128), jnp.int32),
            jax.ShapeDtypeStruct((t, 128), jnp.int32),
            jax.ShapeDtypeStruct((nb, 128), jnp.int32),
        ),
        interpret=_INTERPRET,
    )(tokens, wgp)

    pos2 = pos_out[:, :_K].reshape(p)
    packed = packed_out[:, :_K].reshape(p)
    blk_e = blk_out[:, 0]
    table = jnp.full((nr,), (t - 1) * 65536, jnp.int32).at[pos2].set(packed)

    b1r = b1.reshape(_E, 1, _F)
    b2r = b2.reshape(_E, 1, dv)
    lngr = ln_g.reshape(1, dv)
    lnbr = ln_b.reshape(1, dv)

    grid_spec = pltpu.PrefetchScalarGridSpec(
        num_scalar_prefetch=2,
        grid=(nb,),
        in_specs=[
            pl.BlockSpec((t, dv), lambda i, tb, be: (0, 0)),
            pl.BlockSpec((1, dv, _F), lambda i, tb, be: (be[i], 0, 0)),
            pl.BlockSpec((1, 1, _F), lambda i, tb, be: (be[i], 0, 0)),
            pl.BlockSpec((1, _F, dv), lambda i, tb, be: (be[i], 0, 0)),
            pl.BlockSpec((1, 1, dv), lambda i, tb, be: (be[i], 0, 0)),
            pl.BlockSpec((1, dv), lambda i, tb, be: (0, 0)),
            pl.BlockSpec((1, dv), lambda i, tb, be: (0, 0)),
        ],
        out_specs=pl.BlockSpec((t, dv), lambda i, tb, be: (0, 0)),
        scratch_shapes=[
            pltpu.VMEM((_BLK, dv), jnp.float32),
            pltpu.VMEM((_BLK, dv), jnp.float32),
        ],
    )
    acc = pl.pallas_call(
        functools.partial(_ffn_body, nb=nb, t=t),
        grid_spec=grid_spec,
        out_shape=jax.ShapeDtypeStruct((t, dv), jnp.float32),
        compiler_params=pltpu.CompilerParams(
            dimension_semantics=("arbitrary",)),
        interpret=_INTERPRET,
    )(table, blk_e,
      tokens, W1, b1r, W2, b2r, lngr, lnbr)

    return acc.reshape(bv, nv, dv)


# speedup vs baseline: 1.3646x; 1.0265x over previous
import functools

import jax
import jax.numpy as jnp
from jax.experimental import pallas as pl
from jax.experimental.pallas import tpu as pltpu

_E = 8
_K = 2
_D = 768
_F = 3072
_BLK = 192
_GQ = 65535.0
_INTERPRET = False


def _router_body(tok_ref, wg_ref, pos_ref, packed_ref, blk_ref, *, nb):
    logits = jnp.dot(tok_ref[...], wg_ref[...], preferred_element_type=jnp.float32)
    t, c = logits.shape
    col = jax.lax.broadcasted_iota(jnp.int32, (t, c), 1)
    lg = jnp.where(col < _E, logits, -1e30)
    m1 = jnp.max(lg, axis=1, keepdims=True)
    i1 = jnp.min(jnp.where(lg == m1, col, c), axis=1, keepdims=True)
    lg2 = jnp.where(col == i1, -1e30, lg)
    m2 = jnp.max(lg2, axis=1, keepdims=True)
    i2 = jnp.min(jnp.where(lg2 == m2, col, c), axis=1, keepdims=True)
    g1 = 1.0 / (1.0 + jnp.exp(m2 - m1))

    row = jax.lax.broadcasted_iota(jnp.int32, (t, c), 0)
    q1 = jnp.round(g1 * _GQ).astype(jnp.int32)
    q2 = jnp.round((1.0 - g1) * _GQ).astype(jnp.int32)
    packed_ref[...] = row * 65536 + jnp.where(col == 0, q1,
                                              jnp.where(col == 1, q2, 0))

    oh2 = ((col == i1) | (col == i2)).astype(jnp.float32)
    counts = jnp.sum(oh2, axis=0, keepdims=True)
    nblk = jnp.floor((counts + (_BLK - 1)) * (1.0 / _BLK))
    lane_r = jax.lax.broadcasted_iota(jnp.int32, (c, c), 0)
    lane_c = jax.lax.broadcasted_iota(jnp.int32, (c, c), 1)
    upper = (lane_r < lane_c).astype(jnp.float32)
    blk_start = jnp.dot(nblk, upper, preferred_element_type=jnp.float32)
    off = blk_start * float(_BLK)

    inc = oh2
    k = 1
    while k < t:
        shifted = jnp.concatenate(
            [jnp.zeros((k, c), jnp.float32), inc[: t - k]], axis=0)
        inc = inc + shifted
        k *= 2
    rank = inc - oh2

    sel1 = (col == i1).astype(jnp.float32)
    sel2 = (col == i2).astype(jnp.float32)
    offb = jnp.broadcast_to(off, (t, c))
    pos1 = jnp.sum(sel1 * (offb + rank), axis=1, keepdims=True)
    pos2 = jnp.sum(sel2 * (offb + rank), axis=1, keepdims=True)
    pos_ref[...] = jnp.where(col == 0, pos1, jnp.where(col == 1, pos2, 0.0)
                             ).astype(jnp.int32)

    incl = (blk_start + nblk).astype(jnp.int32)
    jrow = jax.lax.broadcasted_iota(jnp.int32, (nb, c), 0)
    inclb = jnp.broadcast_to(incl, (nb, c))
    lane2 = jax.lax.broadcasted_iota(jnp.int32, (nb, c), 1)
    hit = ((inclb <= jrow) & (lane2 < _E)).astype(jnp.int32)
    blk_ref[...] = jnp.minimum(jnp.sum(hit, axis=1, keepdims=True), _E - 1
                               ) + jnp.zeros((nb, c), jnp.int32)


def _ffn_body(tbl_ref, blke_ref,
              tok_ref, w1_ref, b1_ref, w2_ref, b2_ref, lng_ref, lnb_ref,
              acc_ref, xb_ref, yb_ref, *, nb, t):
    i = pl.program_id(0)

    @pl.when(i == 0)
    def _():
        acc_ref[...] = tok_ref[...]

    def gather(r, _):
        tk = jax.lax.shift_right_logical(tbl_ref[i * _BLK + r], 16)
        xb_ref[r, :] = tok_ref[tk, :]
        return 0

    jax.lax.fori_loop(0, _BLK, gather, 0, unroll=8)

    h = jnp.dot(xb_ref[...], w1_ref[0], preferred_element_type=jnp.float32)
    h = jnp.maximum(h + b1_ref[0], 0.0)
    y = jnp.dot(h, w2_ref[0], preferred_element_type=jnp.float32) + b2_ref[0]
    yb_ref[...] = y

    def scatter(r, _):
        v = tbl_ref[i * _BLK + r]
        tk = jax.lax.shift_right_logical(v, 16)
        g = (v & 65535).astype(jnp.float32) * (1.0 / _GQ)
        acc_ref[tk, :] = acc_ref[tk, :] + g * yb_ref[r, :]
        return 0

    jax.lax.fori_loop(0, _BLK, scatter, 0, unroll=8)

    @pl.when(i == nb - 1)
    def _():
        a = acc_ref[...]
        m = jnp.mean(a, axis=1, keepdims=True)
        v = jnp.mean((a - m) * (a - m), axis=1, keepdims=True)
        acc_ref[...] = (a - m) * jax.lax.rsqrt(v + 1e-5) * lng_ref[...] + lnb_ref[...]


def kernel(x, Wg, W1, b1, W2, b2, ln_g, ln_b):
    bv, nv, dv = x.shape
    t = bv * nv
    p = t * _K
    nb = (p + _BLK - 1) // _BLK + (_E - 1)
    nr = nb * _BLK

    tokens = x.reshape(t, dv)

    wgp = jnp.zeros((dv, 128), Wg.dtype).at[:, :_E].set(Wg)
    pos_out, packed_out, blk_out = pl.pallas_call(
        functools.partial(_router_body, nb=nb),
        out_shape=(
            jax.ShapeDtypeStruct((t, 128), jnp.int32),
            jax.ShapeDtypeStruct((t, 128), jnp.int32),
            jax.ShapeDtypeStruct((nb, 128), jnp.int32),
        ),
        interpret=_INTERPRET,
    )(tokens, wgp)

    pos2 = pos_out[:, :_K].reshape(p)
    packed = packed_out[:, :_K].reshape(p)
    blk_e = blk_out[:, 0]
    table = jnp.full((nr,), (t - 1) * 65536, jnp.int32).at[pos2].set(packed)

    b1r = b1.reshape(_E, 1, _F)
    b2r = b2.reshape(_E, 1, dv)
    lngr = ln_g.reshape(1, dv)
    lnbr = ln_b.reshape(1, dv)

    grid_spec = pltpu.PrefetchScalarGridSpec(
        num_scalar_prefetch=2,
        grid=(nb,),
        in_specs=[
            pl.BlockSpec((t, dv), lambda i, tb, be: (0, 0)),
            pl.BlockSpec((1, dv, _F), lambda i, tb, be: (be[i], 0, 0)),
            pl.BlockSpec((1, 1, _F), lambda i, tb, be: (be[i], 0, 0)),
            pl.BlockSpec((1, _F, dv), lambda i, tb, be: (be[i], 0, 0)),
            pl.BlockSpec((1, 1, dv), lambda i, tb, be: (be[i], 0, 0)),
            pl.BlockSpec((1, dv), lambda i, tb, be: (0, 0)),
            pl.BlockSpec((1, dv), lambda i, tb, be: (0, 0)),
        ],
        out_specs=pl.BlockSpec((t, dv), lambda i, tb, be: (0, 0)),
        scratch_shapes=[
            pltpu.VMEM((_BLK, dv), jnp.float32),
            pltpu.VMEM((_BLK, dv), jnp.float32),
        ],
    )
    acc = pl.pallas_call(
        functools.partial(_ffn_body, nb=nb, t=t),
        grid_spec=grid_spec,
        out_shape=jax.ShapeDtypeStruct((t, dv), jnp.float32),
        compiler_params=pltpu.CompilerParams(
            dimension_semantics=("arbitrary",)),
        interpret=_INTERPRET,
    )(table, blk_e,
      tokens, W1, b1r, W2, b2r, lngr, lnbr)

    return acc.reshape(bv, nv, dv)
